# baseline (speedup 1.0000x reference)
import jax
import jax.numpy as jnp
from jax.experimental import pallas as pl
from jax.experimental.pallas import tpu as pltpu

E = 8
TOP_K = 3
L0 = 187
B = 128
BBLK = 32
NBLK = B // BBLK


def _conv3(a_pad, w, b):
    n, lp2, cin = a_pad.shape
    l = lp2 - 2
    cout = w.shape[2]
    acc = jnp.zeros((n * l, cout), jnp.float32)
    for k in range(3):
        xk = a_pad[:, k:k + l, :].reshape(n * l, cin)
        acc = acc + jnp.dot(xk, w[k], preferred_element_type=jnp.float32)
    return (acc + b).reshape(n, l, cout)


def _pad_l(a):
    return jnp.pad(a, ((0, 0), (1, 1), (0, 0)))


def _pool2(a):
    n, l, c = a.shape
    m = (l - 2) // 2 + 1
    return jnp.max(a[:, :2 * m, :].reshape(n, m, 2, c), axis=2)


def _routing_body(xp3_ref, w1_ref, b1_ref, noise_ref, rw_ref, rb_ref,
                  h_ref, gd_ref, cv_ref):
    xp3 = xp3_ref[:]
    h = (xp3[:, 0:L0, :] * w1_ref[0:1, 0:1, :]
         + xp3[:, 1:L0 + 1, :] * w1_ref[1:2, 0:1, :]
         + xp3[:, 2:L0 + 2, :] * w1_ref[2:3, 0:1, :]
         + b1_ref[0:1, 0:1, :])
    h = jnp.maximum(h, 0.0)
    h_ref[:] = jnp.pad(h, ((0, 0), (1, 1), (0, 0)))
    pooled = jnp.mean(h, axis=1) + noise_ref[:]
    logits = jnp.dot(pooled, rw_ref[:],
                     preferred_element_type=jnp.float32) + rb_ref[:]
    z = logits - jnp.max(logits, axis=-1, keepdims=True)
    ez = jnp.exp(z)
    probs = ez / jnp.sum(ez, axis=-1, keepdims=True)

    i8 = jax.lax.broadcasted_iota(jnp.int32, (B, E), 1)
    p = probs
    gates = []
    onehots = []
    for _ in range(TOP_K):
        g = jnp.max(p, axis=-1, keepdims=True)
        idx = jnp.min(jnp.where(p == g, i8, E), axis=-1, keepdims=True)
        oh = (i8 == idx)
        gates.append(g)
        onehots.append(oh)
        p = jnp.where(oh, -1.0, p)
    gsum = gates[0] + gates[1] + gates[2]
    gd = jnp.zeros((B, E), jnp.float32)
    for s in range(TOP_K):
        gd = gd + jnp.where(onehots[s], gates[s] / gsum, 0.0)
    gd_ref[:] = gd

    mp = jnp.mean(probs, axis=0, keepdims=True)
    m = jnp.mean(mp)
    var = jnp.sum((mp - m) ** 2) / (E - 1)
    cv = jnp.sqrt(var) / (m + 1e-10)
    cv_ref[:, :] = jnp.full((1, 1), cv * cv, jnp.float32)


def _expert_body(h_ref, gd_ref,
                 w11_ref, b11_ref, w12_ref, b12_ref,
                 w21_ref, b21_ref, w22_ref, b22_ref,
                 w31_ref, b31_ref, w32_ref, b32_ref,
                 out_ref, acc_ref):
    e = pl.program_id(0)
    j = pl.program_id(1)

    @pl.when((e == 0) & (j == 0))
    def _init():
        acc_ref[:] = jnp.zeros_like(acc_ref)

    a = h_ref[pl.ds(j * BBLK, BBLK)]
    a = _conv3(a, w11_ref[0], b11_ref[0])
    a = _conv3(_pad_l(a), w12_ref[0], b12_ref[0])
    a = _pool2(jnp.maximum(a, 0.0))
    a = _conv3(_pad_l(a), w21_ref[0], b21_ref[0])
    a = _conv3(_pad_l(a), w22_ref[0], b22_ref[0])
    a = _pool2(jnp.maximum(a, 0.0))
    a = _conv3(_pad_l(a), w31_ref[0], b31_ref[0])
    a = _conv3(_pad_l(a), w32_ref[0], b32_ref[0])
    a = _pool2(jnp.maximum(a, 0.0))

    g = gd_ref[0, pl.ds(j * BBLK, BBLK), :]
    g3 = g.reshape(BBLK, 1, 512)
    cur = acc_ref[pl.ds(j * BBLK, BBLK), 1:24, :]
    acc_ref[pl.ds(j * BBLK, BBLK), 1:24, :] = cur + a * g3

    @pl.when((e == E - 1) & (j == NBLK - 1))
    def _flush():
        out_ref[:] = acc_ref[:]


def _head_body(a_ref, w2_ref, b2_ref, f1_ref, f1b_ref, f2_ref, f2b_ref,
               out_ref):
    a = a_ref[:]
    t = _conv3(a, w2_ref[:], b2_ref[:])
    t = _pool2(jnp.maximum(t, 0.0))
    y1 = jnp.zeros((B, 256), jnp.float32)
    for l in range(11):
        y1 = y1 + jnp.dot(t[:, l, :], f1_ref[l],
                          preferred_element_type=jnp.float32)
    y1 = jnp.maximum(y1 + f1b_ref[:], 0.0)
    out_ref[:] = (jnp.dot(y1, f2_ref[:], preferred_element_type=jnp.float32)
                  + f2b_ref[:])


def kernel(x, params):
    f32 = jnp.float32
    xs = x[:, 0, :]
    xp = jnp.pad(xs, ((0, 0), (1, 1)))
    xp3 = jnp.broadcast_to(xp[:, :, None], (B, L0 + 2, 16))
    w1 = params['conv1_w'][:, 0, :].T.reshape(3, 1, 16)
    b1 = params['conv1_b'].reshape(1, 1, 16)
    noise = jax.random.normal(jax.random.key(1), (B, 16), f32) * 0.05
    rw = params['router_w'].T
    rb = params['router_b'].reshape(1, E)

    h_pad, gd, cv2 = pl.pallas_call(
        _routing_body,
        out_shape=(
            jax.ShapeDtypeStruct((B, L0 + 2, 16), f32),
            jax.ShapeDtypeStruct((B, E), f32),
            jax.ShapeDtypeStruct((1, 1), f32),
        ),
    )(xp3, w1, b1, noise, rw, rb)

    ep = params['experts']
    ws = {k: v.transpose(0, 3, 2, 1) for k, v in ep.items() if k.endswith('_w')}
    bs = {k: v[:, None, :] for k, v in ep.items() if k.endswith('_b')}
    gd_b = jnp.broadcast_to(gd.T.reshape(E, B, 1), (E, B, 512))

    wspec = lambda shape: pl.BlockSpec(shape, lambda e, j: (e, 0, 0, 0))
    bspec = lambda c: pl.BlockSpec((1, 1, c), lambda e, j: (e, 0, 0))

    def full(shape):
        nz = len(shape)
        return pl.BlockSpec(shape, lambda e, j, _n=nz: (0,) * _n)

    eo_pad = pl.pallas_call(
        _expert_body,
        grid=(E, NBLK),
        in_specs=[
            full((B, L0 + 2, 16)),
            pl.BlockSpec((1, B, 512), lambda e, j: (e, 0, 0)),
            wspec((1, 3, 16, 16)), bspec(16),
            wspec((1, 3, 16, 32)), bspec(32),
            wspec((1, 3, 32, 64)), bspec(64),
            wspec((1, 3, 64, 128)), bspec(128),
            wspec((1, 3, 128, 256)), bspec(256),
            wspec((1, 3, 256, 512)), bspec(512),
        ],
        out_specs=full((B, 25, 512)),
        out_shape=jax.ShapeDtypeStruct((B, 25, 512), f32),
        scratch_shapes=[pltpu.VMEM((B, 25, 512), f32)],
    )(h_pad, gd_b,
      ws['b1c1_w'], bs['b1c1_b'], ws['b1c2_w'], bs['b1c2_b'],
      ws['b2c1_w'], bs['b2c1_b'], ws['b2c2_w'], bs['b2c2_b'],
      ws['b3c1_w'], bs['b3c1_b'], ws['b3c2_w'], bs['b3c2_b'])

    w2 = params['conv2_w'].transpose(2, 1, 0)
    b2 = params['conv2_b'][None, :]
    f1 = params['fc1_w'].reshape(256, 1024, 11).transpose(2, 1, 0)
    f1b = params['fc1_b'][None, :]
    f2 = jnp.pad(params['fc2_w'].T, ((0, 0), (0, 3)))
    f2b = jnp.pad(params['fc2_b'], (0, 3))[None, :]

    y8 = pl.pallas_call(
        _head_body,
        out_shape=jax.ShapeDtypeStruct((B, 8), f32),
    )(eo_pad, w2, b2, f1, f1b, f2, f2b)

    return y8[:, :5], cv2[0, 0]


# speedup vs baseline: 1.5760x; 1.5760x over previous
import jax
import jax.numpy as jnp
from jax.experimental import pallas as pl
from jax.experimental.pallas import tpu as pltpu

E = 8
TOP_K = 3
L0 = 187
B = 128
BBLK = 32
NBLK = B // BBLK
bf16 = jnp.bfloat16


def _conv_cat(a_pad, w, b):
    n, lp2, cin = a_pad.shape
    l = lp2 - 2
    xc = jnp.concatenate([a_pad[:, k:k + l, :] for k in range(3)], axis=-1)
    acc = jnp.dot(xc.reshape(n * l, 3 * cin), w,
                  preferred_element_type=jnp.float32)
    return (acc + b).reshape(n, l, w.shape[1])


def _conv_tap(a_pad, w, b):
    n, lp2, cin = a_pad.shape
    l = lp2 - 2
    cout = w.shape[2]
    acc = jnp.zeros((n * l, cout), jnp.float32)
    for k in range(3):
        xk = a_pad[:, k:k + l, :].reshape(n * l, cin)
        acc = acc + jnp.dot(xk, w[k], preferred_element_type=jnp.float32)
    return (acc + b).reshape(n, l, cout)


def _pad_l(a):
    return jnp.pad(a, ((0, 0), (1, 1), (0, 0)))


def _pool2(a):
    n, l, c = a.shape
    m = (l - 2) // 2 + 1
    return jnp.max(a[:, :2 * m, :].reshape(n, m, 2, c), axis=2)


def _routing_body(xp3_ref, w1_ref, b1_ref, noise_ref, rw_ref, rb_ref,
                  h_ref, idx_ref, g_ref, cv_ref):
    xp3 = xp3_ref[:]
    h = (xp3[:, 0:L0, :] * w1_ref[0:1, 0:1, :]
         + xp3[:, 1:L0 + 1, :] * w1_ref[1:2, 0:1, :]
         + xp3[:, 2:L0 + 2, :] * w1_ref[2:3, 0:1, :]
         + b1_ref[0:1, 0:1, :])
    h = jnp.maximum(h, 0.0)
    h_ref[:] = jnp.pad(h, ((0, 0), (1, 1), (0, 0))).astype(bf16)
    pooled = jnp.mean(h, axis=1) + noise_ref[:]
    logits = jnp.dot(pooled, rw_ref[:],
                     preferred_element_type=jnp.float32) + rb_ref[:]
    z = logits - jnp.max(logits, axis=-1, keepdims=True)
    ez = jnp.exp(z)
    probs = ez / jnp.sum(ez, axis=-1, keepdims=True)

    i8 = jax.lax.broadcasted_iota(jnp.int32, (B, E), 1)
    p = probs
    gates = []
    idxs = []
    for _ in range(TOP_K):
        g = jnp.max(p, axis=-1, keepdims=True)
        idx = jnp.min(jnp.where(p == g, i8, E), axis=-1, keepdims=True)
        gates.append(g)
        idxs.append(idx)
        p = jnp.where(i8 == idx, -1.0, p)
    gsum = gates[0] + gates[1] + gates[2]
    idx_ref[:] = jnp.concatenate(idxs, axis=-1)
    g_ref[:] = jnp.concatenate([g / gsum for g in gates], axis=-1)

    mp = jnp.mean(probs, axis=0, keepdims=True)
    m = jnp.mean(mp)
    var = jnp.sum((mp - m) ** 2) / (E - 1)
    cv = jnp.sqrt(var) / (m + 1e-10)
    cv_ref[:, :] = jnp.full((1, 1), cv * cv, jnp.float32)


def _dispatch_body(idx_ref, g_ref, cnt_ref, lst_ref, glst_ref):
    for e in range(E):
        cnt_ref[e] = 0

    def body(b, _):
        for s in range(TOP_K):
            e = idx_ref[b, s]
            c = cnt_ref[e]
            lst_ref[e * B + c] = b
            glst_ref[e * B + c] = g_ref[b, s]
            cnt_ref[e] = c + 1
        return 0

    jax.lax.fori_loop(0, B, body, 0)


def _expert_body(cnt_ref, lst_ref, glst_ref, h_ref,
                 w11_ref, b11_ref, w12_ref, b12_ref,
                 w21_ref, b21_ref, w22_ref, b22_ref,
                 w31_ref, b31_ref, w32_ref, b32_ref,
                 out_ref, gbuf_ref):
    e = pl.program_id(0)
    j = pl.program_id(1)

    @pl.when((e == 0) & (j == 0))
    def _init():
        out_ref[:] = jnp.zeros_like(out_ref)

    n_e = cnt_ref[e]
    base = e * B + j * BBLK

    @pl.when(j * BBLK < n_e)
    def _compute():
        for i in range(BBLK):
            raw = lst_ref[base + i]
            valid = (j * BBLK + i) < n_e
            bi = jnp.clip(jnp.where(valid, raw, 0), 0, B - 1)
            gbuf_ref[i] = h_ref[bi]

        a = gbuf_ref[:]
        a = _conv_cat(a, w11_ref[0], b11_ref[0]).astype(bf16)
        a = _conv_cat(_pad_l(a), w12_ref[0], b12_ref[0])
        a = _pool2(jnp.maximum(a, 0.0).astype(bf16))
        a = _conv_cat(_pad_l(a), w21_ref[0], b21_ref[0]).astype(bf16)
        a = _conv_cat(_pad_l(a), w22_ref[0], b22_ref[0])
        a = _pool2(jnp.maximum(a, 0.0).astype(bf16))
        a = _conv_tap(_pad_l(a), w31_ref[0], b31_ref[0]).astype(bf16)
        a = _conv_tap(_pad_l(a), w32_ref[0], b32_ref[0])
        a = _pool2(jnp.maximum(a, 0.0))

        for i in range(BBLK):
            raw = lst_ref[base + i]
            valid = (j * BBLK + i) < n_e
            bi = jnp.clip(jnp.where(valid, raw, 0), 0, B - 1)
            g = jnp.where(valid, glst_ref[base + i], 0.0)
            out_ref[bi, 1:24, :] = out_ref[bi, 1:24, :] + a[i] * g


def _head_body(a_ref, w2_ref, b2_ref, f1_ref, f1b_ref, f2_ref, f2b_ref,
               out_ref):
    a = a_ref[:].astype(bf16)
    t = _conv_tap(a, w2_ref[:], b2_ref[:])
    t = _pool2(jnp.maximum(t, 0.0).astype(bf16))
    y1 = jnp.zeros((B, 256), jnp.float32)
    for l in range(11):
        y1 = y1 + jnp.dot(t[:, l, :], f1_ref[l],
                          preferred_element_type=jnp.float32)
    y1 = jnp.maximum(y1 + f1b_ref[:], 0.0)
    out_ref[:] = (jnp.dot(y1, f2_ref[:], preferred_element_type=jnp.float32)
                  + f2b_ref[:])


def kernel(x, params):
    f32 = jnp.float32
    xs = x[:, 0, :]
    xp = jnp.pad(xs, ((0, 0), (1, 1)))
    xp3 = jnp.broadcast_to(xp[:, :, None], (B, L0 + 2, 16))
    w1 = params['conv1_w'][:, 0, :].T.reshape(3, 1, 16)
    b1 = params['conv1_b'].reshape(1, 1, 16)
    noise = jax.random.normal(jax.random.key(1), (B, 16), f32) * 0.05
    rw = params['router_w'].T
    rb = params['router_b'].reshape(1, E)

    h_pad, idx3, g3, cv2 = pl.pallas_call(
        _routing_body,
        out_shape=(
            jax.ShapeDtypeStruct((B, L0 + 2, 16), bf16),
            jax.ShapeDtypeStruct((B, TOP_K), jnp.int32),
            jax.ShapeDtypeStruct((B, TOP_K), f32),
            jax.ShapeDtypeStruct((1, 1), f32),
        ),
    )(xp3, w1, b1, noise, rw, rb)

    smem = pl.BlockSpec(memory_space=pltpu.SMEM)
    cnt, lst, glst = pl.pallas_call(
        _dispatch_body,
        in_specs=[smem, smem],
        out_specs=(smem, smem, smem),
        out_shape=(
            jax.ShapeDtypeStruct((E,), jnp.int32),
            jax.ShapeDtypeStruct((E * B,), jnp.int32),
            jax.ShapeDtypeStruct((E * B,), f32),
        ),
    )(idx3, g3)

    ep = params['experts']
    wcat = {k: v.transpose(0, 3, 2, 1).reshape(v.shape[0], 3 * v.shape[2],
                                               v.shape[1]).astype(bf16)
            for k, v in ep.items() if k.endswith('_w')}
    wtap = {k: v.transpose(0, 3, 2, 1).astype(bf16)
            for k, v in ep.items() if k.endswith('_w')}
    bs = {k: v[:, None, :] for k, v in ep.items() if k.endswith('_b')}

    bspec = lambda c: pl.BlockSpec((1, 1, c), lambda e, j: (e, 0, 0))

    eo_pad = pl.pallas_call(
        _expert_body,
        grid=(E, NBLK),
        in_specs=[
            smem, smem, smem,
            pl.BlockSpec((B, L0 + 2, 16), lambda e, j: (0, 0, 0)),
            pl.BlockSpec((1, 48, 16), lambda e, j: (e, 0, 0)), bspec(16),
            pl.BlockSpec((1, 48, 32), lambda e, j: (e, 0, 0)), bspec(32),
            pl.BlockSpec((1, 96, 64), lambda e, j: (e, 0, 0)), bspec(64),
            pl.BlockSpec((1, 192, 128), lambda e, j: (e, 0, 0)), bspec(128),
            pl.BlockSpec((1, 3, 128, 256), lambda e, j: (e, 0, 0, 0)), bspec(256),
            pl.BlockSpec((1, 3, 256, 512), lambda e, j: (e, 0, 0, 0)), bspec(512),
        ],
        out_specs=pl.BlockSpec((B, 25, 512), lambda e, j: (0, 0, 0)),
        out_shape=jax.ShapeDtypeStruct((B, 25, 512), f32),
        scratch_shapes=[pltpu.VMEM((BBLK, L0 + 2, 16), bf16)],
    )(cnt, lst, glst, h_pad,
      wcat['b1c1_w'], bs['b1c1_b'], wcat['b1c2_w'], bs['b1c2_b'],
      wcat['b2c1_w'], bs['b2c1_b'], wcat['b2c2_w'], bs['b2c2_b'],
      wtap['b3c1_w'], bs['b3c1_b'], wtap['b3c2_w'], bs['b3c2_b'])

    w2 = params['conv2_w'].transpose(2, 1, 0).astype(bf16)
    b2 = params['conv2_b'][None, :]
    f1 = params['fc1_w'].reshape(256, 1024, 11).transpose(2, 1, 0).astype(bf16)
    f1b = params['fc1_b'][None, :]
    f2 = jnp.pad(params['fc2_w'].T, ((0, 0), (0, 3)))
    f2b = jnp.pad(params['fc2_b'], (0, 3))[None, :]

    y8 = pl.pallas_call(
        _head_body,
        out_shape=jax.ShapeDtypeStruct((B, 8), f32),
    )(eo_pad, w2, b2, f1, f1b, f2, f2b)

    return y8[:, :5], cv2[0, 0]


# speedup vs baseline: 2.6395x; 1.6748x over previous
import jax
import jax.numpy as jnp
from jax.experimental import pallas as pl
from jax.experimental.pallas import tpu as pltpu

E = 8
TOP_K = 3
L0 = 187
B = 128
BBLK = 32
NBLK = B // BBLK
bf16 = jnp.bfloat16
f32 = jnp.float32


def _pad_l(a):
    return jnp.pad(a, ((0, 0), (1, 1), (0, 0)))


def _pool2(a):
    n, l, c = a.shape
    m = (l - 2) // 2 + 1
    return jnp.max(a[:, :2 * m, :].reshape(n, m, 2, c), axis=2)


def _imcat(a_pad):
    l = a_pad.shape[1] - 2
    return jnp.concatenate([a_pad[:, k:k + l, :] for k in range(3)], axis=-1)


def _routing_body(xp3_ref, w1_ref, b1_ref, noise_ref, rw_ref, rb_ref,
                  h_ref, idx_ref, g_ref, cv_ref):
    xp3 = xp3_ref[:]
    h = (xp3[:, 0:L0, :] * w1_ref[0:1, 0:1, :]
         + xp3[:, 1:L0 + 1, :] * w1_ref[1:2, 0:1, :]
         + xp3[:, 2:L0 + 2, :] * w1_ref[2:3, 0:1, :]
         + b1_ref[0:1, 0:1, :])
    h = jnp.maximum(h, 0.0)
    h_ref[:] = jnp.pad(h, ((0, 0), (1, 1), (0, 0))).astype(bf16)
    pooled = jnp.mean(h, axis=1) + noise_ref[:]
    logits = jnp.dot(pooled, rw_ref[:],
                     preferred_element_type=f32) + rb_ref[:]
    z = logits - jnp.max(logits, axis=-1, keepdims=True)
    ez = jnp.exp(z)
    probs = ez / jnp.sum(ez, axis=-1, keepdims=True)

    i8 = jax.lax.broadcasted_iota(jnp.int32, (B, E), 1)
    p = probs
    gates = []
    idxs = []
    for _ in range(TOP_K):
        g = jnp.max(p, axis=-1, keepdims=True)
        idx = jnp.min(jnp.where(p == g, i8, E), axis=-1, keepdims=True)
        gates.append(g)
        idxs.append(idx)
        p = jnp.where(i8 == idx, -1.0, p)
    gsum = gates[0] + gates[1] + gates[2]
    idx_ref[:] = jnp.concatenate(idxs, axis=-1)
    g_ref[:] = jnp.concatenate([g / gsum for g in gates], axis=-1)

    mp = jnp.mean(probs, axis=0, keepdims=True)
    m = jnp.mean(mp)
    var = jnp.sum((mp - m) ** 2) / (E - 1)
    cv = jnp.sqrt(var) / (m + 1e-10)
    cv_ref[:, :] = jnp.full((1, 1), cv * cv, f32)


def _dispatch_body(idx_ref, g_ref, cnt_ref, lst_ref, glst_ref):
    for e in range(E):
        cnt_ref[e] = 0

    def body(b, _):
        for s in range(TOP_K):
            e = idx_ref[b, s]
            c = cnt_ref[e]
            lst_ref[e * B + c] = b
            glst_ref[e * B + c] = g_ref[b, s]
            cnt_ref[e] = c + 1
        return 0

    jax.lax.fori_loop(0, B, body, 0)


BE = 32
NBE = B // BE


def _early_body(h_ref, w1_ref, bb1_ref, w2_ref, bb2_ref, w3_ref, bb3_ref,
                w4_ref, bb4_ref, out_ref):
    x = h_ref[:]
    t = jnp.dot(_imcat(x).reshape(BE * L0, 48), w1_ref[:],
                preferred_element_type=f32)
    t = ((t + bb1_ref[:]).astype(bf16)).reshape(BE, L0, 128)
    tp = _pad_l(t)
    u = jnp.zeros((BE * L0, 256), f32)
    for k in range(3):
        u = u + jnp.dot(tp[:, k:k + L0, :].reshape(BE * L0, 128), w2_ref[k],
                        preferred_element_type=f32)
    u = jnp.maximum(u + bb2_ref[:], 0.0).astype(bf16).reshape(BE, L0, 256)
    u = _pool2(u)
    up = _pad_l(u)
    v = jnp.zeros((BE * 93, 512), f32)
    for k in range(3):
        v = v + jnp.dot(up[:, k:k + 93, :].reshape(BE * 93, 256), w3_ref[k],
                        preferred_element_type=f32)
    v = (v + bb3_ref[:]).astype(bf16).reshape(BE, 93, 512)
    vp = _pad_l(v)
    out_ref[:] = jnp.zeros_like(out_ref)
    for e in range(E):
        xe = vp[:, :, e * 64:(e + 1) * 64]
        w = jnp.dot(_imcat(xe).reshape(BE * 93, 192), w4_ref[e],
                    preferred_element_type=f32)
        w = jnp.maximum(w + bb4_ref[e], 0.0).astype(bf16).reshape(BE, 93, 128)
        out_ref[e, :, 1:47, :] = _pool2(w)


def _late_body(cnt_ref, lst_ref, glst_ref, d_ref,
               w5_ref, b5_ref, w6_ref, b6_ref,
               out_ref, gbuf_ref):
    e = pl.program_id(0)
    j = pl.program_id(1)

    @pl.when((e == 0) & (j == 0))
    def _init():
        out_ref[:] = jnp.zeros_like(out_ref)

    n_e = cnt_ref[e]
    base = e * B + j * BBLK

    @pl.when(j * BBLK < n_e)
    def _compute():
        for i in range(BBLK):
            raw = lst_ref[base + i]
            valid = (j * BBLK + i) < n_e
            bi = jnp.clip(jnp.where(valid, raw, 0), 0, B - 1)
            gbuf_ref[i] = d_ref[0, bi]

        a = gbuf_ref[:]
        t = jnp.dot(_imcat(a).reshape(BBLK * 46, 384), w5_ref[0],
                    preferred_element_type=f32)
        t = (t + b5_ref[0]).astype(bf16).reshape(BBLK, 46, 256)
        tp = _pad_l(t)
        u = jnp.zeros((BBLK * 46, 512), f32)
        for k in range(3):
            u = u + jnp.dot(tp[:, k:k + 46, :].reshape(BBLK * 46, 256),
                            w6_ref[0, k], preferred_element_type=f32)
        u = jnp.maximum(u + b6_ref[0], 0.0).reshape(BBLK, 46, 512)
        a_out = _pool2(u)

        for i in range(BBLK):
            raw = lst_ref[base + i]
            valid = (j * BBLK + i) < n_e
            bi = jnp.clip(jnp.where(valid, raw, 0), 0, B - 1)
            g = jnp.where(valid, glst_ref[base + i], 0.0)
            out_ref[bi, 1:24, :] = out_ref[bi, 1:24, :] + a_out[i] * g


def _head_body(a_ref, w2_ref, b2_ref, f1_ref, f1b_ref, f2_ref, f2b_ref,
               out_ref):
    a = a_ref[:].astype(bf16)
    t = jnp.zeros((B * 23, 1024), f32)
    for k in range(3):
        t = t + jnp.dot(a[:, k:k + 23, :].reshape(B * 23, 512), w2_ref[k],
                        preferred_element_type=f32)
    t = jnp.maximum(t + b2_ref[:], 0.0).astype(bf16).reshape(B, 23, 1024)
    t = _pool2(t)
    y1 = jnp.zeros((B, 256), f32)
    for l in range(11):
        y1 = y1 + jnp.dot(t[:, l, :], f1_ref[l],
                          preferred_element_type=f32)
    y1 = jnp.maximum(y1 + f1b_ref[:], 0.0)
    out_ref[:] = (jnp.dot(y1, f2_ref[:], preferred_element_type=f32)
                  + f2b_ref[:])


def kernel(x, params):
    xs = x[:, 0, :]
    xp = jnp.pad(xs, ((0, 0), (1, 1)))
    xp3 = jnp.broadcast_to(xp[:, :, None], (B, L0 + 2, 16))
    w1r = params['conv1_w'][:, 0, :].T.reshape(3, 1, 16)
    b1r = params['conv1_b'].reshape(1, 1, 16)
    noise = jax.random.normal(jax.random.key(1), (B, 16), f32) * 0.05
    rw = params['router_w'].T
    rb = params['router_b'].reshape(1, E)

    h_pad, idx3, g3, cv2 = pl.pallas_call(
        _routing_body,
        out_shape=(
            jax.ShapeDtypeStruct((B, L0 + 2, 16), bf16),
            jax.ShapeDtypeStruct((B, TOP_K), jnp.int32),
            jax.ShapeDtypeStruct((B, TOP_K), f32),
            jax.ShapeDtypeStruct((1, 1), f32),
        ),
    )(xp3, w1r, b1r, noise, rw, rb)

    smem = pl.BlockSpec(memory_space=pltpu.SMEM)
    cnt, lst, glst = pl.pallas_call(
        _dispatch_body,
        in_specs=[smem, smem],
        out_specs=(smem, smem, smem),
        out_shape=(
            jax.ShapeDtypeStruct((E,), jnp.int32),
            jax.ShapeDtypeStruct((E * B,), jnp.int32),
            jax.ShapeDtypeStruct((E * B,), f32),
        ),
    )(idx3, g3)

    ep = params['experts']
    eye = jnp.eye(E, dtype=f32)

    def bd(w):
        wt = w.transpose(3, 0, 2, 1)
        out = jnp.einsum('ef,keio->keifo', eye, wt)
        k, e_, ci, f_, co = out.shape
        return out.reshape(k, e_ * ci, f_ * co).astype(bf16)

    def cat_rows(w):
        return w.transpose(0, 3, 2, 1).reshape(
            w.shape[0], 3 * w.shape[2], w.shape[1]).astype(bf16)

    w1c = cat_rows(ep['b1c1_w']).transpose(1, 0, 2).reshape(48, E * 16)
    bb1 = ep['b1c1_b'].reshape(1, E * 16)
    w2bd = bd(ep['b1c2_w'])
    bb2 = ep['b1c2_b'].reshape(1, E * 32)
    w3bd = bd(ep['b2c1_w'])
    bb3 = ep['b2c1_b'].reshape(1, E * 64)
    w4c = cat_rows(ep['b2c2_w'])
    bb4 = ep['b2c2_b'][:, None, :]

    def _const(shape):
        nz = len(shape)
        return pl.BlockSpec(shape, lambda jb, _n=nz: (0,) * _n)

    dense4 = pl.pallas_call(
        _early_body,
        grid=(NBE,),
        in_specs=[
            pl.BlockSpec((BE, L0 + 2, 16), lambda jb: (jb, 0, 0)),
            _const((48, 128)), _const((1, 128)),
            _const((3, 128, 256)), _const((1, 256)),
            _const((3, 256, 512)), _const((1, 512)),
            _const((E, 192, 128)), _const((E, 1, 128)),
        ],
        out_specs=pl.BlockSpec((E, BE, 48, 128), lambda jb: (0, jb, 0, 0)),
        out_shape=jax.ShapeDtypeStruct((E, B, 48, 128), bf16),
    )(h_pad, w1c, bb1, w2bd, bb2, w3bd, bb3, w4c, bb4)

    w5c = cat_rows(ep['b3c1_w'])
    b5 = ep['b3c1_b'][:, None, :]
    w6t = ep['b3c2_w'].transpose(0, 3, 2, 1).astype(bf16)
    b6 = ep['b3c2_b'][:, None, :]

    eo_pad = pl.pallas_call(
        _late_body,
        grid=(E, NBLK),
        in_specs=[
            smem, smem, smem,
            pl.BlockSpec((1, B, 48, 128), lambda e, j: (e, 0, 0, 0)),
            pl.BlockSpec((1, 384, 256), lambda e, j: (e, 0, 0)),
            pl.BlockSpec((1, 1, 256), lambda e, j: (e, 0, 0)),
            pl.BlockSpec((1, 3, 256, 512), lambda e, j: (e, 0, 0, 0)),
            pl.BlockSpec((1, 1, 512), lambda e, j: (e, 0, 0)),
        ],
        out_specs=pl.BlockSpec((B, 25, 512), lambda e, j: (0, 0, 0)),
        out_shape=jax.ShapeDtypeStruct((B, 25, 512), f32),
        scratch_shapes=[pltpu.VMEM((BBLK, 48, 128), bf16)],
    )(cnt, lst, glst, dense4, w5c, b5, w6t, b6)

    w2 = params['conv2_w'].transpose(2, 1, 0).astype(bf16)
    b2 = params['conv2_b'][None, :]
    f1 = params['fc1_w'].reshape(256, 1024, 11).transpose(2, 1, 0).astype(bf16)
    f1b = params['fc1_b'][None, :]
    f2 = jnp.pad(params['fc2_w'].T, ((0, 0), (0, 3)))
    f2b = jnp.pad(params['fc2_b'], (0, 3))[None, :]

    y8 = pl.pallas_call(
        _head_body,
        out_shape=jax.ShapeDtypeStruct((B, 8), f32),
    )(eo_pad, w2, b2, f1, f1b, f2, f2b)

    return y8[:, :5], cv2[0, 0]


# speedup vs baseline: 2.7507x; 1.0422x over previous
import jax
import jax.numpy as jnp
from jax.experimental import pallas as pl
from jax.experimental.pallas import tpu as pltpu

E = 8
TOP_K = 3
L0 = 187
B = 128
BBLK = 32
NBLK = B // BBLK
bf16 = jnp.bfloat16
f32 = jnp.float32


def _pad_l(a):
    return jnp.pad(a, ((0, 0), (1, 1), (0, 0)))


def _pool2(a):
    n, l, c = a.shape
    m = (l - 2) // 2 + 1
    return jnp.max(a[:, :2 * m, :].reshape(n, m, 2, c), axis=2)


def _imcat(a_pad):
    l = a_pad.shape[1] - 2
    return jnp.concatenate([a_pad[:, k:k + l, :] for k in range(3)], axis=-1)


def _routing_body(xp3_ref, w1_ref, b1_ref, noise_ref, rw_ref, rb_ref,
                  h_ref, idx_ref, g_ref, cv_ref):
    xp3 = xp3_ref[:].astype(bf16).astype(f32)
    wr = w1_ref[:].astype(bf16).astype(f32)
    h = (xp3[:, 0:L0, :] * wr[0:1, 0:1, :]
         + xp3[:, 1:L0 + 1, :] * wr[1:2, 0:1, :]
         + xp3[:, 2:L0 + 2, :] * wr[2:3, 0:1, :]
         + b1_ref[0:1, 0:1, :])
    h = jnp.maximum(h, 0.0)
    h_ref[:] = jnp.pad(h, ((0, 0), (1, 1), (0, 0))).astype(bf16)
    pooled = jnp.mean(h, axis=1) + noise_ref[:]
    logits = jnp.dot(pooled, rw_ref[:],
                     preferred_element_type=f32) + rb_ref[:]
    z = logits - jnp.max(logits, axis=-1, keepdims=True)
    ez = jnp.exp(z)
    probs = ez / jnp.sum(ez, axis=-1, keepdims=True)

    i8 = jax.lax.broadcasted_iota(jnp.int32, (B, E), 1)
    p = probs
    gates = []
    idxs = []
    for _ in range(TOP_K):
        g = jnp.max(p, axis=-1, keepdims=True)
        idx = jnp.min(jnp.where(p == g, i8, E), axis=-1, keepdims=True)
        gates.append(g)
        idxs.append(idx)
        p = jnp.where(i8 == idx, -1.0, p)
    gsum = gates[0] + gates[1] + gates[2]
    idx_ref[:] = jnp.concatenate(idxs, axis=-1)
    g_ref[:] = jnp.concatenate([g / gsum for g in gates], axis=-1)

    mp = jnp.mean(probs, axis=0, keepdims=True)
    m = jnp.mean(mp)
    var = jnp.sum((mp - m) ** 2) / (E - 1)
    cv = jnp.sqrt(var) / (m + 1e-10)
    cv_ref[:, :] = jnp.full((1, 1), cv * cv, f32)


def _dispatch_body(idx_ref, g_ref, cnt_ref, lst_ref, glst_ref):
    for e in range(E):
        cnt_ref[e] = 0

    def body(b, _):
        for s in range(TOP_K):
            e = idx_ref[b, s]
            c = cnt_ref[e]
            lst_ref[e * B + c] = b
            glst_ref[e * B + c] = g_ref[b, s]
            cnt_ref[e] = c + 1
        return 0

    jax.lax.fori_loop(0, B, body, 0)


BE = 32
NBE = B // BE


def _early_body(h_ref, w1_ref, bb1_ref, w2_ref, bb2_ref, w3_ref, bb3_ref,
                w4_ref, bb4_ref, out_ref):
    x = h_ref[:]
    t = jnp.dot(_imcat(x).reshape(BE * L0, 48), w1_ref[:],
                preferred_element_type=f32)
    t = (t + bb1_ref[:]).astype(bf16).reshape(BE, L0, 128)
    tp = _pad_l(t)
    u = jnp.dot(_imcat(tp).reshape(BE * L0, 384), w2_ref[:],
                preferred_element_type=f32)
    u = jnp.maximum(u + bb2_ref[:], 0.0).astype(bf16).reshape(BE, L0, 256)
    u = _pool2(u)
    up = _pad_l(u)
    v = jnp.dot(_imcat(up).reshape(BE * 93, 768), w3_ref[:],
                preferred_element_type=f32)
    v = (v + bb3_ref[:]).astype(bf16).reshape(BE, 93, 512)
    vp = _pad_l(v)
    out_ref[:] = jnp.zeros_like(out_ref)
    for e in range(E):
        xe = vp[:, :, e * 64:(e + 1) * 64]
        w = jnp.dot(_imcat(xe).reshape(BE * 93, 192), w4_ref[e],
                    preferred_element_type=f32)
        w = jnp.maximum(w + bb4_ref[e], 0.0).astype(bf16).reshape(BE, 93, 128)
        out_ref[e, :, 1:47, :] = _pool2(w)


def _late_body(cnt_ref, lst_ref, glst_ref, d_ref,
               w5_ref, b5_ref, w6_ref, b6_ref,
               out_ref, gbuf_ref):
    e = pl.program_id(0)
    j = pl.program_id(1)

    @pl.when((e == 0) & (j == 0))
    def _init():
        out_ref[:] = jnp.zeros_like(out_ref)

    n_e = cnt_ref[e]
    base = e * B + j * BBLK

    @pl.when(j * BBLK < n_e)
    def _compute():
        for i in range(BBLK):
            raw = lst_ref[base + i]
            valid = (j * BBLK + i) < n_e
            bi = jnp.clip(jnp.where(valid, raw, 0), 0, B - 1)
            gbuf_ref[i] = d_ref[0, bi]

        a = gbuf_ref[:]
        t = jnp.dot(_imcat(a).reshape(BBLK * 46, 384), w5_ref[0],
                    preferred_element_type=f32)
        t = (t + b5_ref[0]).astype(bf16).reshape(BBLK, 46, 256)
        tp = _pad_l(t)
        u = jnp.dot(_imcat(tp).reshape(BBLK * 46, 768), w6_ref[0],
                    preferred_element_type=f32)
        u = jnp.maximum(u + b6_ref[0], 0.0).reshape(BBLK, 46, 512)
        a_out = _pool2(u)

        for i in range(BBLK):
            raw = lst_ref[base + i]
            valid = (j * BBLK + i) < n_e
            bi = jnp.clip(jnp.where(valid, raw, 0), 0, B - 1)
            g = jnp.where(valid, glst_ref[base + i], 0.0)
            out_ref[bi, 1:24, :] = out_ref[bi, 1:24, :] + a_out[i] * g


def _head_body(a_ref, w2_ref, b2_ref, f1_ref, f1b_ref, f2_ref, f2b_ref,
               out_ref):
    a = a_ref[:].astype(bf16)
    t = jnp.dot(_imcat(a).reshape(B * 23, 1536), w2_ref[:],
                preferred_element_type=f32)
    t = jnp.maximum(t + b2_ref[:], 0.0).astype(bf16).reshape(B, 23, 1024)
    t = _pool2(t)
    y1 = jnp.zeros((B, 256), f32)
    for l in range(11):
        y1 = y1 + jnp.dot(t[:, l, :], f1_ref[l],
                          preferred_element_type=f32)
    y1 = jnp.maximum(y1 + f1b_ref[:], 0.0)
    out_ref[:] = (jnp.dot(y1, f2_ref[:], preferred_element_type=f32)
                  + f2b_ref[:])


def kernel(x, params):
    xs = x[:, 0, :]
    xp = jnp.pad(xs, ((0, 0), (1, 1)))
    xp3 = jnp.broadcast_to(xp[:, :, None], (B, L0 + 2, 16))
    w1r = params['conv1_w'][:, 0, :].T.reshape(3, 1, 16)
    b1r = params['conv1_b'].reshape(1, 1, 16)
    noise = jax.random.normal(jax.random.key(1), (B, 16), f32) * 0.05
    rw = params['router_w'].T
    rb = params['router_b'].reshape(1, E)

    h_pad, idx3, g3, cv2 = pl.pallas_call(
        _routing_body,
        out_shape=(
            jax.ShapeDtypeStruct((B, L0 + 2, 16), bf16),
            jax.ShapeDtypeStruct((B, TOP_K), jnp.int32),
            jax.ShapeDtypeStruct((B, TOP_K), f32),
            jax.ShapeDtypeStruct((1, 1), f32),
        ),
    )(xp3, w1r, b1r, noise, rw, rb)

    smem = pl.BlockSpec(memory_space=pltpu.SMEM)
    cnt, lst, glst = pl.pallas_call(
        _dispatch_body,
        in_specs=[smem, smem],
        out_specs=(smem, smem, smem),
        out_shape=(
            jax.ShapeDtypeStruct((E,), jnp.int32),
            jax.ShapeDtypeStruct((E * B,), jnp.int32),
            jax.ShapeDtypeStruct((E * B,), f32),
        ),
    )(idx3, g3)

    ep = params['experts']
    eye = jnp.eye(E, dtype=f32)

    def bd(w):
        wt = w.transpose(3, 0, 2, 1)
        out = jnp.einsum('ef,keio->keifo', eye, wt)
        k, e_, ci, f_, co = out.shape
        return out.reshape(k, e_ * ci, f_ * co).astype(bf16)

    def cat_rows(w):
        return w.transpose(0, 3, 2, 1).reshape(
            w.shape[0], 3 * w.shape[2], w.shape[1]).astype(bf16)

    w1c = cat_rows(ep['b1c1_w']).transpose(1, 0, 2).reshape(48, E * 16)
    bb1 = ep['b1c1_b'].reshape(1, E * 16)
    w2bd = bd(ep['b1c2_w']).reshape(3 * 128, 256)
    bb2 = ep['b1c2_b'].reshape(1, E * 32)
    w3bd = bd(ep['b2c1_w']).reshape(3 * 256, 512)
    bb3 = ep['b2c1_b'].reshape(1, E * 64)
    w4c = cat_rows(ep['b2c2_w'])
    bb4 = ep['b2c2_b'][:, None, :]

    def _const(shape):
        nz = len(shape)
        return pl.BlockSpec(shape, lambda jb, _n=nz: (0,) * _n)

    dense4 = pl.pallas_call(
        _early_body,
        grid=(NBE,),
        in_specs=[
            pl.BlockSpec((BE, L0 + 2, 16), lambda jb: (jb, 0, 0)),
            _const((48, 128)), _const((1, 128)),
            _const((384, 256)), _const((1, 256)),
            _const((768, 512)), _const((1, 512)),
            _const((E, 192, 128)), _const((E, 1, 128)),
        ],
        out_specs=pl.BlockSpec((E, BE, 48, 128), lambda jb: (0, jb, 0, 0)),
        out_shape=jax.ShapeDtypeStruct((E, B, 48, 128), bf16),
    )(h_pad, w1c, bb1, w2bd, bb2, w3bd, bb3, w4c, bb4)

    w5c = cat_rows(ep['b3c1_w'])
    b5 = ep['b3c1_b'][:, None, :]
    w6t = cat_rows(ep['b3c2_w'])
    b6 = ep['b3c2_b'][:, None, :]

    eo_pad = pl.pallas_call(
        _late_body,
        grid=(E, NBLK),
        in_specs=[
            smem, smem, smem,
            pl.BlockSpec((1, B, 48, 128), lambda e, j: (e, 0, 0, 0)),
            pl.BlockSpec((1, 384, 256), lambda e, j: (e, 0, 0)),
            pl.BlockSpec((1, 1, 256), lambda e, j: (e, 0, 0)),
            pl.BlockSpec((1, 768, 512), lambda e, j: (e, 0, 0)),
            pl.BlockSpec((1, 1, 512), lambda e, j: (e, 0, 0)),
        ],
        out_specs=pl.BlockSpec((B, 25, 512), lambda e, j: (0, 0, 0)),
        out_shape=jax.ShapeDtypeStruct((B, 25, 512), f32),
        scratch_shapes=[pltpu.VMEM((BBLK, 48, 128), bf16)],
    )(cnt, lst, glst, dense4, w5c, b5, w6t, b6)

    w2 = params['conv2_w'].transpose(2, 1, 0).astype(bf16).reshape(1536, 1024)
    b2 = params['conv2_b'][None, :]
    f1 = params['fc1_w'].reshape(256, 1024, 11).transpose(2, 1, 0).astype(bf16)
    f1b = params['fc1_b'][None, :]
    f2 = jnp.pad(params['fc2_w'].T, ((0, 0), (0, 3)))
    f2b = jnp.pad(params['fc2_b'], (0, 3))[None, :]

    y8 = pl.pallas_call(
        _head_body,
        out_shape=jax.ShapeDtypeStruct((B, 8), f32),
    )(eo_pad, w2, b2, f1, f1b, f2, f2b)

    return y8[:, :5], cv2[0, 0]
